# baseline (device time: 9743 ns/iter reference)
import jax
import jax.numpy as jnp
from jax import lax
from jax.experimental import pallas as pl
from jax.experimental.pallas import tpu as pltpu

N_DEV = 4
N_TILES = 8


def kernel(x):
    m_per, n = x.shape
    tile = m_per // N_TILES

    def body(x_hbm, out_ref, buf_ref, copy_sems, mine_ref, comm_ref,
             send_sems, recv_sems):
        my = lax.axis_index("i")

        barrier_sem = pltpu.get_barrier_semaphore()
        for d in range(1, N_DEV):
            peer = lax.rem(my + d, N_DEV)
            pl.semaphore_signal(
                barrier_sem,
                inc=1,
                device_id=(peer,),
                device_id_type=pl.DeviceIdType.MESH,
            )

        copies = []
        for t in range(N_TILES):
            cp = pltpu.make_async_copy(
                x_hbm.at[pl.ds(t * tile, tile), :],
                buf_ref.at[t],
                copy_sems.at[t],
            )
            cp.start()
            copies.append(cp)

        acc = jnp.zeros((1, n), dtype=x_hbm.dtype)
        for t in range(N_TILES):
            copies[t].wait()
            acc = acc + jnp.sum(buf_ref[t], axis=0, keepdims=True)
        mine_ref[:, :] = acc

        pl.semaphore_wait(barrier_sem, N_DEV - 1)

        rdmas = []
        for d in range(1, N_DEV):
            peer = lax.rem(my + d, N_DEV)
            s = d - 1
            rdma = pltpu.make_async_remote_copy(
                src_ref=mine_ref,
                dst_ref=comm_ref.at[s],
                send_sem=send_sems.at[s],
                recv_sem=recv_sems.at[s],
                device_id=(peer,),
                device_id_type=pl.DeviceIdType.MESH,
            )
            rdma.start()
            rdmas.append(rdma)

        for rdma in rdmas:
            rdma.wait()

        out_ref[:, :] = (
            mine_ref[:, :]
            + comm_ref[0, :, :]
            + comm_ref[1, :, :]
            + comm_ref[2, :, :]
        )

    return pl.pallas_call(
        body,
        out_shape=jax.ShapeDtypeStruct((1, n), x.dtype),
        in_specs=[pl.BlockSpec(memory_space=pl.ANY)],
        out_specs=pl.BlockSpec(memory_space=pltpu.VMEM),
        scratch_shapes=[
            pltpu.VMEM((N_TILES, tile, n), x.dtype),
            pltpu.SemaphoreType.DMA((N_TILES,)),
            pltpu.VMEM((1, n), x.dtype),
            pltpu.VMEM((N_DEV - 1, 1, n), x.dtype),
            pltpu.SemaphoreType.DMA((N_DEV - 1,)),
            pltpu.SemaphoreType.DMA((N_DEV - 1,)),
        ],
        compiler_params=pltpu.CompilerParams(collective_id=0),
    )(x)
